# baseline (speedup 1.0000x reference)
import jax
import jax.numpy as jnp
from jax.experimental import pallas as pl
from jax.experimental.pallas import tpu as pltpu

_BLOCK = 1024


def _body(ei_ref, rw_ref, se_ref, h_ref, o_ref):
    ei = ei_ref[0]
    a = jnp.where(se_ref[...] == ei, rw_ref[...], 0.0)
    ones = jnp.ones((2, 128), jnp.float32)
    w = jax.lax.dot_general(a, ones, (((0,), (0,)), ((), ())),
                            preferred_element_type=jnp.float32)
    o_ref[...] = h_ref[...] * w[:, 0:1]


def kernel(routing_weights, selected_experts, hidden_state, expert_idx):
    n, k = routing_weights.shape
    d = hidden_state.shape[1]
    ei = jnp.asarray(expert_idx, jnp.int32).reshape((1,))
    rw_t = routing_weights.T
    se_t = selected_experts.astype(jnp.int32).T
    grid = (n // _BLOCK,)
    return pl.pallas_call(
        _body,
        grid=grid,
        in_specs=[
            pl.BlockSpec(memory_space=pltpu.SMEM),
            pl.BlockSpec((k, _BLOCK), lambda i: (0, i)),
            pl.BlockSpec((k, _BLOCK), lambda i: (0, i)),
            pl.BlockSpec((_BLOCK, d), lambda i: (i, 0)),
        ],
        out_specs=pl.BlockSpec((_BLOCK, d), lambda i: (i, 0)),
        out_shape=jax.ShapeDtypeStruct((n, d), hidden_state.dtype),
        compiler_params=pltpu.CompilerParams(
            dimension_semantics=("parallel",)),
    )(ei, rw_t, se_t, hidden_state)
